# Initial kernel scaffold; baseline (speedup 1.0000x reference)
import jax
import jax.numpy as jnp
from jax.experimental import pallas as pl


def kernel(h, edge_index, edge_weight, protein_h, W1r, b1r, W2r, b2r, Wg0, bg0, Wg1, bg1, W1p, b1p, W2p, b2p):
    raise NotImplementedError("write your pallas kernel here")


# speedup vs baseline: 11.3962x; 11.3962x over previous
import functools

import jax
import jax.numpy as jnp
from jax import lax
from jax.experimental import pallas as pl
from jax.experimental.pallas import tpu as pltpu
from jax.experimental.pallas import tpu_sc as plsc

_N = 10000
_E = 320000
_DIN = 128
_HID = 256
_OUT = 128
_PIN = 1280
_NPROT = 19

_NC = 2
_NS = 16
_NW = _NC * _NS
_EPW = _E // _NW
_CH = 80
_NCHUNK = _EPW // _CH
_NPAD = 10240
_RPT = _NPAD // _NS
_ZCH = 128
_DEGPAD = 10240
_DEGTOT = 2 * _DEGPAD
_DPT = _DEGPAD // _NS
_SCH = 25
_NSUP = _NCHUNK // _SCH
_EPS = _SCH * _CH

_F32 = jnp.float32
_HIGH = lax.Precision.HIGHEST

def _mesh():
    return plsc.VectorSubcoreMesh(core_axis_name="c", subcore_axis_name="s",
                                  num_cores=_NC, num_subcores=_NS)


def _degree_body(ei_hbm, ones_hbm, zeros_hbm, out_hbm, idx_v, ones_v, buf_v,
                 dego_sh, degi_sh, sem0, sem1):
    cid = lax.axis_index("c")
    sid = lax.axis_index("s")
    wid = sid * _NC + cid

    pltpu.sync_copy(ei_hbm.at[0, wid], idx_v.at[0])
    pltpu.sync_copy(ei_hbm.at[1, wid], idx_v.at[1])
    pltpu.sync_copy(ones_hbm, ones_v)

    pltpu.sync_copy(zeros_hbm.at[pl.ds(sid * _DPT, _DPT)], buf_v)
    pltpu.sync_copy(buf_v, dego_sh.at[pl.ds(sid * _DPT, _DPT)])
    pltpu.sync_copy(buf_v, degi_sh.at[pl.ds(sid * _DPT, _DPT)])
    plsc.subcore_barrier()

    def _scat_s(s, _):
        def _scat(j, _):
            c0 = pltpu.async_copy(ones_v, dego_sh.at[idx_v.at[0, s, j]],
                                  sem0, add=True)
            c1 = pltpu.async_copy(ones_v, degi_sh.at[idx_v.at[1, s, j]],
                                  sem1, add=True)
            c0.wait()
            c1.wait()
            return ()
        lax.fori_loop(0, _SCH, _scat, ())
        return ()
    lax.fori_loop(0, _NSUP, _scat_s, ())
    plsc.subcore_barrier()

    pltpu.sync_copy(dego_sh.at[pl.ds(sid * _DPT, _DPT)], buf_v)
    pltpu.sync_copy(buf_v, out_hbm.at[cid, 0, pl.ds(sid * _DPT, _DPT)])
    pltpu.sync_copy(degi_sh.at[pl.ds(sid * _DPT, _DPT)], buf_v)
    pltpu.sync_copy(buf_v, out_hbm.at[cid, 1, pl.ds(sid * _DPT, _DPT)])


def _degrees(ei_r, ones80, zeros1d):
    return pl.kernel(
        _degree_body,
        out_type=jax.ShapeDtypeStruct((_NC, 2, _DEGPAD), _F32),
        mesh=_mesh(),
        compiler_params=pltpu.CompilerParams(needs_layout_passes=False),
        scratch_types=[
            pltpu.VMEM((2, _NSUP, _SCH, _CH), jnp.int32),
            pltpu.VMEM((_CH,), _F32),
            pltpu.VMEM((_DPT,), _F32),
            pltpu.VMEM_SHARED((_DEGPAD,), _F32),
            pltpu.VMEM_SHARED((_DEGPAD,), _F32),
            pltpu.SemaphoreType.DMA,
            pltpu.SemaphoreType.DMA,
        ],
    )(ei_r, ones80, zeros1d)


def _norm_body(deg_ref, norms_ref):
    d = deg_ref[0] + deg_ref[1]
    norms_ref[...] = lax.rsqrt(jnp.maximum(d, 1.0))


def _norms(deg):
    return pl.pallas_call(
        _norm_body,
        out_shape=jax.ShapeDtypeStruct((2, _DEGPAD), _F32),
    )(deg)


_BD_ROWS = 1000
_BD_GRID = _N // _BD_ROWS


def _dense_body(h_ref, w1_ref, b1_ref, w2_ref, b2_ref, wg_ref, no_ref,
                zs_ref, wf_ref, bf_ref):
    i = pl.program_id(0)

    @pl.when(i == 0)
    def _():
        wf_ref[...] = jnp.dot(w2_ref[...], wg_ref[...],
                              preferred_element_type=_F32, precision=_HIGH)
        bf_ref[...] = jnp.dot(b2_ref[...], wg_ref[...],
                              preferred_element_type=_F32, precision=_HIGH)

    hm = jnp.dot(h_ref[...], w1_ref[...], preferred_element_type=_F32,
                 precision=_HIGH) + b1_ref[...]
    hm = jnp.maximum(hm, 0.0)
    z = jnp.dot(hm, wf_ref[...], preferred_element_type=_F32,
                precision=_HIGH) + bf_ref[...]
    zs_ref[...] = z * no_ref[...].reshape(_BD_ROWS, 1)


def _dense(h, W1r, b1r, W2r, b2r, Wg1, norms3):
    return pl.pallas_call(
        _dense_body,
        grid=(_BD_GRID,),
        in_specs=[
            pl.BlockSpec((_BD_ROWS, _DIN), lambda i: (i, 0)),
            pl.BlockSpec((_DIN, _HID), lambda i: (0, 0)),
            pl.BlockSpec((1, _HID), lambda i: (0, 0)),
            pl.BlockSpec((_HID, _OUT), lambda i: (0, 0)),
            pl.BlockSpec((1, _OUT), lambda i: (0, 0)),
            pl.BlockSpec((_OUT, _OUT), lambda i: (0, 0)),
            pl.BlockSpec((1, _BD_ROWS, 1), lambda i: (0, i, 0)),
        ],
        out_specs=pl.BlockSpec((_BD_ROWS, _OUT), lambda i: (i, 0)),
        out_shape=jax.ShapeDtypeStruct((_N, _OUT), _F32),
        scratch_shapes=[
            pltpu.VMEM((_HID, _OUT), _F32),
            pltpu.VMEM((1, _OUT), _F32),
        ],
    )(h, W1r, b1r, W2r, b2r, Wg1, norms3)


def _edge_kernel_body(zs_hbm, ei_hbm, ew_hbm, zeros_hbm, out_hbm, src_i,
                      dst_i, ew_v, rows, agg_sh, gsem):
    cid = lax.axis_index("c")
    sid = lax.axis_index("s")
    wid = sid * _NC + cid

    pltpu.sync_copy(zeros_hbm, agg_sh.at[pl.ds(sid * _RPT, _RPT)])
    plsc.subcore_barrier()

    def _sup(s, _):
        pltpu.sync_copy(ei_hbm.at[0, wid, s], src_i)
        pltpu.sync_copy(ei_hbm.at[1, wid, s], dst_i)
        pltpu.sync_copy(ew_hbm.at[wid, s], ew_v)

        def _chunk(j, _):
            pltpu.async_copy(zs_hbm.at[src_i.at[j]], rows, gsem).wait()

            def _scale(e, _):
                ewv = plsc.load_gather(
                    ew_v, [jnp.full((16,), j * _CH + e, jnp.int32)])
                for g in range(8):
                    sl = pl.ds(g * 16, 16)
                    rows[e, sl] = rows[e, sl] * ewv
                return ()
            lax.fori_loop(0, _CH, _scale, ())

            pltpu.sync_copy(rows, agg_sh.at[dst_i.at[j]], add=True)
            return ()
        lax.fori_loop(0, _SCH, _chunk, ())
        return ()
    lax.fori_loop(0, _NSUP, _sup, ())
    plsc.subcore_barrier()

    pltpu.sync_copy(agg_sh.at[pl.ds(sid * _RPT, _RPT)],
                    out_hbm.at[cid, pl.ds(sid * _RPT, _RPT)])


def _edges(zs, ei_r, ew_r, zeros2d):
    return pl.kernel(
        _edge_kernel_body,
        out_type=jax.ShapeDtypeStruct((_NC, _NPAD, _OUT), _F32),
        mesh=_mesh(),
        compiler_params=pltpu.CompilerParams(needs_layout_passes=False),
        scratch_types=[
            pltpu.VMEM((_SCH, _CH), jnp.int32),
            pltpu.VMEM((_SCH, _CH), jnp.int32),
            pltpu.VMEM((_EPS,), _F32),
            pltpu.VMEM((_CH, _OUT), _F32),
            pltpu.VMEM_SHARED((_NPAD, _OUT), _F32),
            pltpu.SemaphoreType.DMA,
        ],
    )(zs, ei_r, ew_r, zeros2d)


def _final_body(a0_ref, a1_ref, ni_ref, bg_ref, ph_ref, w1p_ref, b1p_ref,
                w2p_ref, b2p_ref, hf_ref, po_ref):
    i = pl.program_id(0)

    a = a0_ref[0] + a1_ref[0]
    hl = a * ni_ref[...].reshape(_BD_ROWS, 1) + bg_ref[...]
    h1 = jnp.where(hl >= 0.0, hl, 0.01 * hl)
    part = jnp.sum(h1, axis=0, keepdims=True)

    @pl.when(i == 0)
    def _():
        hf_ref[...] = jnp.zeros_like(hf_ref)
        hm = jnp.dot(ph_ref[...], w1p_ref[...], preferred_element_type=_F32,
                     precision=_HIGH) + b1p_ref[...]
        hm = jnp.maximum(hm, 0.0)
        po_ref[...] = jnp.dot(hm, w2p_ref[...], preferred_element_type=_F32,
                              precision=_HIGH) + b2p_ref[...]

    hf_ref[...] += part

    @pl.when(i == _BD_GRID - 1)
    def _():
        hf_ref[...] = hf_ref[...] * (1.0 / _N)


def _final(aggs, norms3, bg1, ph, W1p, b1p, W2p, b2p):
    return pl.pallas_call(
        _final_body,
        grid=(_BD_GRID,),
        in_specs=[
            pl.BlockSpec((1, _BD_ROWS, _OUT), lambda i: (0, i, 0)),
            pl.BlockSpec((1, _BD_ROWS, _OUT), lambda i: (1, i, 0)),
            pl.BlockSpec((1, _BD_ROWS, 1), lambda i: (1, i, 0)),
            pl.BlockSpec((1, _OUT), lambda i: (0, 0)),
            pl.BlockSpec((_NPROT, _PIN), lambda i: (0, 0)),
            pl.BlockSpec((_PIN, _HID), lambda i: (0, 0)),
            pl.BlockSpec((1, _HID), lambda i: (0, 0)),
            pl.BlockSpec((_HID, _OUT), lambda i: (0, 0)),
            pl.BlockSpec((1, _OUT), lambda i: (0, 0)),
        ],
        out_specs=[
            pl.BlockSpec((1, _OUT), lambda i: (0, 0)),
            pl.BlockSpec((_NPROT, _OUT), lambda i: (0, 0)),
        ],
        out_shape=[
            jax.ShapeDtypeStruct((1, _OUT), _F32),
            jax.ShapeDtypeStruct((_NPROT, _OUT), _F32),
        ],
    )(aggs, aggs, norms3, bg1, ph, W1p, b1p, W2p, b2p)


def kernel(h, edge_index, edge_weight, protein_h, W1r, b1r, W2r, b2r,
           Wg0, bg0, Wg1, bg1, W1p, b1p, W2p, b2p):
    del Wg0, bg0

    ei_r = edge_index.reshape(2, _NW, _NSUP, _SCH, _CH)
    ew_r = edge_weight.reshape(_NW, _NSUP, _EPS)

    ones80 = jnp.ones((_CH,), _F32)
    zeros1d = jnp.zeros((_DEGPAD,), _F32)
    zeros2d = jnp.zeros((_RPT, _OUT), _F32)

    deg = _degrees(ei_r, ones80, zeros1d)
    norms = _norms(deg)
    norms3 = norms.reshape(2, _DEGPAD, 1)

    zs = _dense(h, W1r, b1r.reshape(1, _HID), W2r, b2r.reshape(1, _OUT),
                Wg1, norms3)
    aggs = _edges(zs, ei_r, ew_r, zeros2d)

    hf, po = _final(aggs, norms3, bg1.reshape(1, _OUT), protein_h,
                    W1p, b1p.reshape(1, _HID), W2p, b2p.reshape(1, _OUT))
    return (hf, po)


# speedup vs baseline: 15.1182x; 1.3266x over previous
import functools

import jax
import jax.numpy as jnp
from jax import lax
from jax.experimental import pallas as pl
from jax.experimental.pallas import tpu as pltpu
from jax.experimental.pallas import tpu_sc as plsc

_N = 10000
_E = 320000
_DIN = 128
_HID = 256
_OUT = 128
_PIN = 1280
_NPROT = 19

_NC = 2
_NS = 16
_NW = _NC * _NS
_EPW = _E // _NW
_CH = 40
_NCHUNK = _EPW // _CH
_NPAD = 10240
_RPT = _NPAD // _NS
_ZCH = 128
_DEGPAD = 10240
_DEGTOT = 2 * _DEGPAD
_DPT = _DEGPAD // _NS
_SCH = 50
_NSUP = _NCHUNK // _SCH
_EPS = _SCH * _CH
_K = 5
_GPS = _SCH // _K
_PADROW = 10016

_F32 = jnp.float32
_HIGH = lax.Precision.HIGHEST

def _mesh():
    return plsc.VectorSubcoreMesh(core_axis_name="c", subcore_axis_name="s",
                                  num_cores=_NC, num_subcores=_NS)


def _degree_body(ei_hbm, ones_hbm, zeros_hbm, gidx_hbm, out_hbm, idx_v,
                 ones_v, buf_v, gidx_v, dego_sh, degi_sh, sem0, sem1, sem2,
                 sem3):
    cid = lax.axis_index("c")
    sid = lax.axis_index("s")
    wid = sid * _NC + cid

    pltpu.sync_copy(ei_hbm.at[0, wid], idx_v.at[0])
    pltpu.sync_copy(ei_hbm.at[1, wid], idx_v.at[1])
    pltpu.sync_copy(ones_hbm, ones_v)
    pltpu.sync_copy(gidx_hbm, gidx_v)

    pltpu.sync_copy(zeros_hbm.at[pl.ds(sid * _DPT, _DPT)], buf_v)
    pltpu.sync_copy(buf_v, dego_sh.at[pl.ds(sid * _DPT, _DPT)])
    pltpu.sync_copy(buf_v, degi_sh.at[pl.ds(sid * _DPT, _DPT)])
    plsc.subcore_barrier()

    pltpu.async_copy(ones_v, dego_sh.at[gidx_v], sem0, add=True)
    pltpu.async_copy(ones_v, degi_sh.at[gidx_v], sem1, add=True)
    pltpu.async_copy(ones_v, dego_sh.at[gidx_v], sem2, add=True)
    pltpu.async_copy(ones_v, degi_sh.at[gidx_v], sem3, add=True)

    def _scat_s(s, _):
        def _scat(jj, _):
            j0 = 2 * jj
            j1 = j0 + 1
            pltpu.make_async_copy(ones_v, dego_sh.at[gidx_v], sem0).wait()
            pltpu.async_copy(ones_v, dego_sh.at[idx_v.at[0, s, j0]], sem0,
                             add=True)
            pltpu.make_async_copy(ones_v, degi_sh.at[gidx_v], sem1).wait()
            pltpu.async_copy(ones_v, degi_sh.at[idx_v.at[1, s, j0]], sem1,
                             add=True)
            pltpu.make_async_copy(ones_v, dego_sh.at[gidx_v], sem2).wait()
            pltpu.async_copy(ones_v, dego_sh.at[idx_v.at[0, s, j1]], sem2,
                             add=True)
            pltpu.make_async_copy(ones_v, degi_sh.at[gidx_v], sem3).wait()
            pltpu.async_copy(ones_v, degi_sh.at[idx_v.at[1, s, j1]], sem3,
                             add=True)
            return ()
        lax.fori_loop(0, _SCH // 2, _scat, ())
        return ()
    lax.fori_loop(0, _NSUP, _scat_s, ())
    pltpu.make_async_copy(ones_v, dego_sh.at[gidx_v], sem0).wait()
    pltpu.make_async_copy(ones_v, degi_sh.at[gidx_v], sem1).wait()
    pltpu.make_async_copy(ones_v, dego_sh.at[gidx_v], sem2).wait()
    pltpu.make_async_copy(ones_v, degi_sh.at[gidx_v], sem3).wait()
    plsc.subcore_barrier()

    pltpu.sync_copy(dego_sh.at[pl.ds(sid * _DPT, _DPT)], buf_v)
    pltpu.sync_copy(buf_v, out_hbm.at[cid, 0, pl.ds(sid * _DPT, _DPT)])
    pltpu.sync_copy(degi_sh.at[pl.ds(sid * _DPT, _DPT)], buf_v)
    pltpu.sync_copy(buf_v, out_hbm.at[cid, 1, pl.ds(sid * _DPT, _DPT)])


def _degrees(ei_r, ones80, zeros1d, gidx):
    return pl.kernel(
        _degree_body,
        out_type=jax.ShapeDtypeStruct((_NC, 2, _DEGPAD), _F32),
        mesh=_mesh(),
        compiler_params=pltpu.CompilerParams(needs_layout_passes=False),
        scratch_types=[
            pltpu.VMEM((2, _NSUP, _SCH, _CH), jnp.int32),
            pltpu.VMEM((_CH,), _F32),
            pltpu.VMEM((_DPT,), _F32),
            pltpu.VMEM((_CH,), jnp.int32),
            pltpu.VMEM_SHARED((_DEGPAD,), _F32),
            pltpu.VMEM_SHARED((_DEGPAD,), _F32),
            pltpu.SemaphoreType.DMA,
            pltpu.SemaphoreType.DMA,
            pltpu.SemaphoreType.DMA,
            pltpu.SemaphoreType.DMA,
        ],
    )(ei_r, ones80, zeros1d, gidx)


def _norm_body(deg_ref, norms_ref):
    d = deg_ref[0] + deg_ref[1]
    norms_ref[...] = lax.rsqrt(jnp.maximum(d, 1.0))


def _norms(deg):
    return pl.pallas_call(
        _norm_body,
        out_shape=jax.ShapeDtypeStruct((2, _DEGPAD), _F32),
    )(deg)


_BD_ROWS = 1000
_BD_GRID = _N // _BD_ROWS


def _dense_body(h_ref, w1_ref, b1_ref, w2_ref, b2_ref, wg_ref, no_ref,
                zs_ref, wf_ref, bf_ref):
    i = pl.program_id(0)

    @pl.when(i == 0)
    def _():
        wf_ref[...] = jnp.dot(w2_ref[...], wg_ref[...],
                              preferred_element_type=_F32, precision=_HIGH)
        bf_ref[...] = jnp.dot(b2_ref[...], wg_ref[...],
                              preferred_element_type=_F32, precision=_HIGH)

    hm = jnp.dot(h_ref[...], w1_ref[...], preferred_element_type=_F32,
                 precision=_HIGH) + b1_ref[...]
    hm = jnp.maximum(hm, 0.0)
    z = jnp.dot(hm, wf_ref[...], preferred_element_type=_F32,
                precision=_HIGH) + bf_ref[...]
    zs_ref[...] = z * no_ref[...].reshape(_BD_ROWS, 1)


def _dense(h, W1r, b1r, W2r, b2r, Wg1, norms3):
    return pl.pallas_call(
        _dense_body,
        grid=(_BD_GRID,),
        in_specs=[
            pl.BlockSpec((_BD_ROWS, _DIN), lambda i: (i, 0)),
            pl.BlockSpec((_DIN, _HID), lambda i: (0, 0)),
            pl.BlockSpec((1, _HID), lambda i: (0, 0)),
            pl.BlockSpec((_HID, _OUT), lambda i: (0, 0)),
            pl.BlockSpec((1, _OUT), lambda i: (0, 0)),
            pl.BlockSpec((_OUT, _OUT), lambda i: (0, 0)),
            pl.BlockSpec((1, _BD_ROWS, 1), lambda i: (0, i, 0)),
        ],
        out_specs=pl.BlockSpec((_BD_ROWS, _OUT), lambda i: (i, 0)),
        out_shape=jax.ShapeDtypeStruct((_N, _OUT), _F32),
        scratch_shapes=[
            pltpu.VMEM((_HID, _OUT), _F32),
            pltpu.VMEM((1, _OUT), _F32),
        ],
    )(h, W1r, b1r, W2r, b2r, Wg1, norms3)


def _edge_kernel_body(zs_hbm, ei_hbm, ew_hbm, zeros_hbm, gidx_hbm, out_hbm,
                      src_i, dst_i, ew_v, rows, gidx_v,
                      agg_sh, gs0, gs1, gs2, gs3, gs4, ss0, ss1, ss2, ss3,
                      ss4):
    cid = lax.axis_index("c")
    sid = lax.axis_index("s")
    wid = sid * _NC + cid
    gs = (gs0, gs1, gs2, gs3, gs4)
    ss = (ss0, ss1, ss2, ss3, ss4)

    pltpu.sync_copy(gidx_hbm, gidx_v)

    pltpu.sync_copy(zeros_hbm, agg_sh.at[pl.ds(sid * _RPT, _RPT)])

    for k in range(_K):
        pltpu.async_copy(rows.at[k], agg_sh.at[gidx_v], ss[k], add=True)
    plsc.subcore_barrier()

    def _sup(s, _):
        pltpu.sync_copy(ei_hbm.at[0, wid, s], src_i)
        pltpu.sync_copy(ei_hbm.at[1, wid, s], dst_i)
        pltpu.sync_copy(ew_hbm.at[wid, s], ew_v)

        def _grp(g, _):
            for k in range(_K):
                j = g * _K + k
                pltpu.make_async_copy(rows.at[k], agg_sh.at[gidx_v],
                                      ss[k]).wait()
                pltpu.async_copy(zs_hbm.at[src_i.at[j]], rows.at[k], gs[k])
            for k in range(_K):
                j = g * _K + k
                pltpu.make_async_copy(zs_hbm.at[src_i.at[j]], rows.at[k],
                                      gs[k]).wait()

                def _scale(e, _, k=k, j=j):
                    ewv = plsc.load_gather(
                        ew_v, [jnp.full((16,), j * _CH + e, jnp.int32)])
                    for gg in range(8):
                        sl = pl.ds(gg * 16, 16)
                        rows[k, e, sl] = rows[k, e, sl] * ewv
                    return ()
                lax.fori_loop(0, _CH, _scale, ())

                pltpu.async_copy(rows.at[k], agg_sh.at[dst_i.at[j]], ss[k],
                                 add=True)
            return ()
        lax.fori_loop(0, _GPS, _grp, ())
        return ()
    lax.fori_loop(0, _NSUP, _sup, ())

    for k in range(_K):
        pltpu.make_async_copy(rows.at[k], agg_sh.at[gidx_v], ss[k]).wait()
    plsc.subcore_barrier()

    pltpu.sync_copy(agg_sh.at[pl.ds(sid * _RPT, _RPT)],
                    out_hbm.at[cid, pl.ds(sid * _RPT, _RPT)])


def _edges(zs, ei_r, ew_r, zeros2d, gidx):
    return pl.kernel(
        _edge_kernel_body,
        out_type=jax.ShapeDtypeStruct((_NC, _NPAD, _OUT), _F32),
        mesh=_mesh(),
        compiler_params=pltpu.CompilerParams(needs_layout_passes=False),
        scratch_types=[
            pltpu.VMEM((_SCH, _CH), jnp.int32),
            pltpu.VMEM((_SCH, _CH), jnp.int32),
            pltpu.VMEM((_EPS,), _F32),
            pltpu.VMEM((_K, _CH, _OUT), _F32),
            pltpu.VMEM((_CH,), jnp.int32),
            pltpu.VMEM_SHARED((_NPAD, _OUT), _F32),
        ] + [pltpu.SemaphoreType.DMA] * (2 * _K),
    )(zs, ei_r, ew_r, zeros2d, gidx)


def _final_body(a0_ref, a1_ref, ni_ref, bg_ref, ph_ref, w1p_ref, b1p_ref,
                w2p_ref, b2p_ref, hf_ref, po_ref):
    i = pl.program_id(0)

    a = a0_ref[0] + a1_ref[0]
    hl = a * ni_ref[...].reshape(_BD_ROWS, 1) + bg_ref[...]
    h1 = jnp.where(hl >= 0.0, hl, 0.01 * hl)
    part = jnp.sum(h1, axis=0, keepdims=True)

    @pl.when(i == 0)
    def _():
        hf_ref[...] = jnp.zeros_like(hf_ref)
        hm = jnp.dot(ph_ref[...], w1p_ref[...], preferred_element_type=_F32,
                     precision=_HIGH) + b1p_ref[...]
        hm = jnp.maximum(hm, 0.0)
        po_ref[...] = jnp.dot(hm, w2p_ref[...], preferred_element_type=_F32,
                              precision=_HIGH) + b2p_ref[...]

    hf_ref[...] += part

    @pl.when(i == _BD_GRID - 1)
    def _():
        hf_ref[...] = hf_ref[...] * (1.0 / _N)


def _final(aggs, norms3, bg1, ph, W1p, b1p, W2p, b2p):
    return pl.pallas_call(
        _final_body,
        grid=(_BD_GRID,),
        in_specs=[
            pl.BlockSpec((1, _BD_ROWS, _OUT), lambda i: (0, i, 0)),
            pl.BlockSpec((1, _BD_ROWS, _OUT), lambda i: (1, i, 0)),
            pl.BlockSpec((1, _BD_ROWS, 1), lambda i: (1, i, 0)),
            pl.BlockSpec((1, _OUT), lambda i: (0, 0)),
            pl.BlockSpec((_NPROT, _PIN), lambda i: (0, 0)),
            pl.BlockSpec((_PIN, _HID), lambda i: (0, 0)),
            pl.BlockSpec((1, _HID), lambda i: (0, 0)),
            pl.BlockSpec((_HID, _OUT), lambda i: (0, 0)),
            pl.BlockSpec((1, _OUT), lambda i: (0, 0)),
        ],
        out_specs=[
            pl.BlockSpec((1, _OUT), lambda i: (0, 0)),
            pl.BlockSpec((_NPROT, _OUT), lambda i: (0, 0)),
        ],
        out_shape=[
            jax.ShapeDtypeStruct((1, _OUT), _F32),
            jax.ShapeDtypeStruct((_NPROT, _OUT), _F32),
        ],
    )(aggs, aggs, norms3, bg1, ph, W1p, b1p, W2p, b2p)


def kernel(h, edge_index, edge_weight, protein_h, W1r, b1r, W2r, b2r,
           Wg0, bg0, Wg1, bg1, W1p, b1p, W2p, b2p):
    del Wg0, bg0

    ei_r = edge_index.reshape(2, _NW, _NSUP, _SCH, _CH)
    ew_r = edge_weight.reshape(_NW, _NSUP, _EPS)

    ones80 = jnp.ones((_CH,), _F32)
    zeros1d = jnp.zeros((_DEGPAD,), _F32)
    zeros2d = jnp.zeros((_RPT, _OUT), _F32)
    gidx = jnp.full((_CH,), _PADROW, jnp.int32)

    deg = _degrees(ei_r, ones80, zeros1d, gidx)
    norms = _norms(deg)
    norms3 = norms.reshape(2, _DEGPAD, 1)

    zs = _dense(h, W1r, b1r.reshape(1, _HID), W2r, b2r.reshape(1, _OUT),
                Wg1, norms3)
    aggs = _edges(zs, ei_r, ew_r, zeros2d, gidx)

    hf, po = _final(aggs, norms3, bg1.reshape(1, _OUT), protein_h,
                    W1p, b1p.reshape(1, _HID), W2p, b2p.reshape(1, _OUT))
    return (hf, po)


# speedup vs baseline: 18.3746x; 1.2154x over previous
import functools

import jax
import jax.numpy as jnp
from jax import lax
from jax.experimental import pallas as pl
from jax.experimental.pallas import tpu as pltpu
from jax.experimental.pallas import tpu_sc as plsc

_N = 10000
_E = 320000
_DIN = 128
_HID = 256
_OUT = 128
_PIN = 1280
_NPROT = 19

_NC = 2
_NS = 16
_NW = _NC * _NS
_EPW = _E // _NW
_CH = 40
_NCHUNK = _EPW // _CH
_NPAD = 10240
_RPT = _NPAD // _NS
_ZCH = 128
_DEGPAD = 10240
_DEGTOT = 2 * _DEGPAD
_DPT = _DEGPAD // _NS
_SCH = 50
_NSUP = _NCHUNK // _SCH
_EPS = _SCH * _CH
_K = 5
_GPS = _SCH // _K
_PADROW = 10016

_F32 = jnp.float32
_HIGH = lax.Precision.HIGHEST

def _mesh():
    return plsc.VectorSubcoreMesh(core_axis_name="c", subcore_axis_name="s",
                                  num_cores=_NC, num_subcores=_NS)


def _degree_body(ei_hbm, ones_hbm, zeros_hbm, gidx_hbm, out_hbm, idx_v,
                 ones_v, buf_v, gidx_v, dego_sh, degi_sh, sem0, sem1, sem2,
                 sem3):
    cid = lax.axis_index("c")
    sid = lax.axis_index("s")
    wid = sid * _NC + cid

    pltpu.sync_copy(ei_hbm.at[0, wid], idx_v.at[0])
    pltpu.sync_copy(ei_hbm.at[1, wid], idx_v.at[1])
    pltpu.sync_copy(ones_hbm, ones_v)
    pltpu.sync_copy(gidx_hbm, gidx_v)

    pltpu.sync_copy(zeros_hbm.at[pl.ds(sid * _DPT, _DPT)], buf_v)
    pltpu.sync_copy(buf_v, dego_sh.at[pl.ds(sid * _DPT, _DPT)])
    pltpu.sync_copy(buf_v, degi_sh.at[pl.ds(sid * _DPT, _DPT)])
    plsc.subcore_barrier()

    pltpu.async_copy(ones_v, dego_sh.at[gidx_v], sem0, add=True)
    pltpu.async_copy(ones_v, degi_sh.at[gidx_v], sem1, add=True)
    pltpu.async_copy(ones_v, dego_sh.at[gidx_v], sem2, add=True)
    pltpu.async_copy(ones_v, degi_sh.at[gidx_v], sem3, add=True)

    def _scat_s(s, _):
        def _scat(jj, _):
            j0 = 2 * jj
            j1 = j0 + 1
            pltpu.make_async_copy(ones_v, dego_sh.at[gidx_v], sem0).wait()
            pltpu.async_copy(ones_v, dego_sh.at[idx_v.at[0, s, j0]], sem0,
                             add=True)
            pltpu.make_async_copy(ones_v, degi_sh.at[gidx_v], sem1).wait()
            pltpu.async_copy(ones_v, degi_sh.at[idx_v.at[1, s, j0]], sem1,
                             add=True)
            pltpu.make_async_copy(ones_v, dego_sh.at[gidx_v], sem2).wait()
            pltpu.async_copy(ones_v, dego_sh.at[idx_v.at[0, s, j1]], sem2,
                             add=True)
            pltpu.make_async_copy(ones_v, degi_sh.at[gidx_v], sem3).wait()
            pltpu.async_copy(ones_v, degi_sh.at[idx_v.at[1, s, j1]], sem3,
                             add=True)
            return ()
        lax.fori_loop(0, _SCH // 2, _scat, ())
        return ()
    lax.fori_loop(0, _NSUP, _scat_s, ())
    pltpu.make_async_copy(ones_v, dego_sh.at[gidx_v], sem0).wait()
    pltpu.make_async_copy(ones_v, degi_sh.at[gidx_v], sem1).wait()
    pltpu.make_async_copy(ones_v, dego_sh.at[gidx_v], sem2).wait()
    pltpu.make_async_copy(ones_v, degi_sh.at[gidx_v], sem3).wait()
    plsc.subcore_barrier()

    pltpu.sync_copy(dego_sh.at[pl.ds(sid * _DPT, _DPT)], buf_v)
    pltpu.sync_copy(buf_v, out_hbm.at[cid, 0, pl.ds(sid * _DPT, _DPT)])
    pltpu.sync_copy(degi_sh.at[pl.ds(sid * _DPT, _DPT)], buf_v)
    pltpu.sync_copy(buf_v, out_hbm.at[cid, 1, pl.ds(sid * _DPT, _DPT)])


def _degrees(ei_r, ones80, zeros1d, gidx):
    return pl.kernel(
        _degree_body,
        out_type=jax.ShapeDtypeStruct((_NC, 2, _DEGPAD), _F32),
        mesh=_mesh(),
        compiler_params=pltpu.CompilerParams(needs_layout_passes=False),
        scratch_types=[
            pltpu.VMEM((2, _NSUP, _SCH, _CH), jnp.int32),
            pltpu.VMEM((_CH,), _F32),
            pltpu.VMEM((_DPT,), _F32),
            pltpu.VMEM((_CH,), jnp.int32),
            pltpu.VMEM_SHARED((_DEGPAD,), _F32),
            pltpu.VMEM_SHARED((_DEGPAD,), _F32),
            pltpu.SemaphoreType.DMA,
            pltpu.SemaphoreType.DMA,
            pltpu.SemaphoreType.DMA,
            pltpu.SemaphoreType.DMA,
        ],
    )(ei_r, ones80, zeros1d, gidx)


_BD_ROWS = 1000
_BD_GRID = _N // _BD_ROWS


def _dense_body(h_ref, w1_ref, b1_ref, w2_ref, b2_ref, wg_ref, deg_ref,
                zs_ref, wf_ref, bf_ref):
    i = pl.program_id(0)

    @pl.when(i == 0)
    def _():
        wf_ref[...] = jnp.dot(w2_ref[...], wg_ref[...],
                              preferred_element_type=_F32, precision=_HIGH)
        bf_ref[...] = jnp.dot(b2_ref[...], wg_ref[...],
                              preferred_element_type=_F32, precision=_HIGH)

    hm = jnp.dot(h_ref[...], w1_ref[...], preferred_element_type=_F32,
                 precision=_HIGH) + b1_ref[...]
    hm = jnp.maximum(hm, 0.0)
    z = jnp.dot(hm, wf_ref[...], preferred_element_type=_F32,
                precision=_HIGH) + bf_ref[...]
    d = deg_ref[0, 0] + deg_ref[1, 0]
    zs_ref[...] = z * lax.rsqrt(jnp.maximum(d, 1.0))


def _dense(h, W1r, b1r, W2r, b2r, Wg1, deg4):
    return pl.pallas_call(
        _dense_body,
        grid=(_BD_GRID,),
        in_specs=[
            pl.BlockSpec((_BD_ROWS, _DIN), lambda i: (i, 0)),
            pl.BlockSpec((_DIN, _HID), lambda i: (0, 0)),
            pl.BlockSpec((1, _HID), lambda i: (0, 0)),
            pl.BlockSpec((_HID, _OUT), lambda i: (0, 0)),
            pl.BlockSpec((1, _OUT), lambda i: (0, 0)),
            pl.BlockSpec((_OUT, _OUT), lambda i: (0, 0)),
            pl.BlockSpec((_NC, 1, _BD_ROWS, 1), lambda i: (0, 0, i, 0)),
        ],
        out_specs=pl.BlockSpec((_BD_ROWS, _OUT), lambda i: (i, 0)),
        out_shape=jax.ShapeDtypeStruct((_N, _OUT), _F32),
        scratch_shapes=[
            pltpu.VMEM((_HID, _OUT), _F32),
            pltpu.VMEM((1, _OUT), _F32),
        ],
    )(h, W1r, b1r, W2r, b2r, Wg1, deg4[:, :1])


def _edge_kernel_body(zs_hbm, ei_hbm, ew_hbm, zeros_hbm, gidx_hbm, out_hbm,
                      src_i, dst_i, ew_v, rows, gidx_v,
                      agg_sh, gs0, gs1, gs2, gs3, gs4, ss0, ss1, ss2, ss3,
                      ss4):
    cid = lax.axis_index("c")
    sid = lax.axis_index("s")
    wid = sid * _NC + cid
    gs = (gs0, gs1, gs2, gs3, gs4)
    ss = (ss0, ss1, ss2, ss3, ss4)

    pltpu.sync_copy(gidx_hbm, gidx_v)

    pltpu.sync_copy(zeros_hbm, agg_sh.at[pl.ds(sid * _RPT, _RPT)])

    for k in range(_K):
        pltpu.async_copy(rows.at[k], agg_sh.at[gidx_v], ss[k], add=True)
    plsc.subcore_barrier()

    def _sup(s, _):
        pltpu.sync_copy(ei_hbm.at[0, wid, s], src_i)
        pltpu.sync_copy(ei_hbm.at[1, wid, s], dst_i)
        pltpu.sync_copy(ew_hbm.at[wid, s], ew_v)

        for k in range(3):
            pltpu.make_async_copy(rows.at[k], agg_sh.at[gidx_v],
                                  ss[k]).wait()
            pltpu.async_copy(zs_hbm.at[src_i.at[k]], rows.at[k], gs[k])

        def _prefetch(g, kp):
            jn = (g + 1) * _K + kp

            @pl.when(g + 1 < _GPS)
            def _():
                pltpu.make_async_copy(rows.at[kp], agg_sh.at[gidx_v],
                                      ss[kp]).wait()
                pltpu.async_copy(zs_hbm.at[src_i.at[jn]], rows.at[kp],
                                 gs[kp])

        def _grp(g, _):
            for k in range(_K):
                j = g * _K + k
                if k < 2:
                    b = k + 3
                    jb = g * _K + b
                    pltpu.make_async_copy(rows.at[b], agg_sh.at[gidx_v],
                                          ss[b]).wait()
                    pltpu.async_copy(zs_hbm.at[src_i.at[jb]], rows.at[b],
                                     gs[b])
                pltpu.make_async_copy(zs_hbm.at[src_i.at[j]], rows.at[k],
                                      gs[k]).wait()

                def _scale(e2, _, k=k, j=j):
                    for u in range(2):
                        e = e2 * 2 + u
                        ewv = plsc.load_gather(
                            ew_v, [jnp.full((16,), j * _CH + e, jnp.int32)])
                        for gg in range(8):
                            sl = pl.ds(gg * 16, 16)
                            rows[k, e, sl] = rows[k, e, sl] * ewv
                    return ()
                lax.fori_loop(0, _CH // 2, _scale, ())

                pltpu.async_copy(rows.at[k], agg_sh.at[dst_i.at[j]], ss[k],
                                 add=True)
                if k >= 2:
                    _prefetch(g, k - 2)
            return ()
        lax.fori_loop(0, _GPS, _grp, ())
        return ()
    lax.fori_loop(0, _NSUP, _sup, ())

    for k in range(_K):
        pltpu.make_async_copy(rows.at[k], agg_sh.at[gidx_v], ss[k]).wait()
    plsc.subcore_barrier()

    pltpu.sync_copy(agg_sh.at[pl.ds(sid * _RPT, _RPT)],
                    out_hbm.at[cid, pl.ds(sid * _RPT, _RPT)])


def _edges(zs, ei_r, ew_r, zeros2d, gidx):
    return pl.kernel(
        _edge_kernel_body,
        out_type=jax.ShapeDtypeStruct((_NC, _NPAD, _OUT), _F32),
        mesh=_mesh(),
        compiler_params=pltpu.CompilerParams(needs_layout_passes=False),
        scratch_types=[
            pltpu.VMEM((_SCH, _CH), jnp.int32),
            pltpu.VMEM((_SCH, _CH), jnp.int32),
            pltpu.VMEM((_EPS,), _F32),
            pltpu.VMEM((_K, _CH, _OUT), _F32),
            pltpu.VMEM((_CH,), jnp.int32),
            pltpu.VMEM_SHARED((_NPAD, _OUT), _F32),
        ] + [pltpu.SemaphoreType.DMA] * (2 * _K),
    )(zs, ei_r, ew_r, zeros2d, gidx)


def _final_body(a0_ref, a1_ref, deg_ref, bg_ref, ph_ref, w1p_ref, b1p_ref,
                w2p_ref, b2p_ref, hf_ref, po_ref):
    i = pl.program_id(0)

    a = a0_ref[0] + a1_ref[0]
    d = deg_ref[0, 0] + deg_ref[1, 0]
    hl = a * lax.rsqrt(jnp.maximum(d, 1.0)) + bg_ref[...]
    h1 = jnp.where(hl >= 0.0, hl, 0.01 * hl)
    part = jnp.sum(h1, axis=0, keepdims=True)

    @pl.when(i == 0)
    def _():
        hf_ref[...] = jnp.zeros_like(hf_ref)
        hm = jnp.dot(ph_ref[...], w1p_ref[...], preferred_element_type=_F32,
                     precision=_HIGH) + b1p_ref[...]
        hm = jnp.maximum(hm, 0.0)
        po_ref[...] = jnp.dot(hm, w2p_ref[...], preferred_element_type=_F32,
                              precision=_HIGH) + b2p_ref[...]

    hf_ref[...] += part

    @pl.when(i == _BD_GRID - 1)
    def _():
        hf_ref[...] = hf_ref[...] * (1.0 / _N)


def _final(aggs, deg4, bg1, ph, W1p, b1p, W2p, b2p):
    return pl.pallas_call(
        _final_body,
        grid=(_BD_GRID,),
        in_specs=[
            pl.BlockSpec((1, _BD_ROWS, _OUT), lambda i: (0, i, 0)),
            pl.BlockSpec((1, _BD_ROWS, _OUT), lambda i: (1, i, 0)),
            pl.BlockSpec((_NC, 1, _BD_ROWS, 1), lambda i: (0, 0, i, 0)),
            pl.BlockSpec((1, _OUT), lambda i: (0, 0)),
            pl.BlockSpec((_NPROT, _PIN), lambda i: (0, 0)),
            pl.BlockSpec((_PIN, _HID), lambda i: (0, 0)),
            pl.BlockSpec((1, _HID), lambda i: (0, 0)),
            pl.BlockSpec((_HID, _OUT), lambda i: (0, 0)),
            pl.BlockSpec((1, _OUT), lambda i: (0, 0)),
        ],
        out_specs=[
            pl.BlockSpec((1, _OUT), lambda i: (0, 0)),
            pl.BlockSpec((_NPROT, _OUT), lambda i: (0, 0)),
        ],
        out_shape=[
            jax.ShapeDtypeStruct((1, _OUT), _F32),
            jax.ShapeDtypeStruct((_NPROT, _OUT), _F32),
        ],
    )(aggs, aggs, deg4[:, 1:], bg1, ph, W1p, b1p, W2p, b2p)


def kernel(h, edge_index, edge_weight, protein_h, W1r, b1r, W2r, b2r,
           Wg0, bg0, Wg1, bg1, W1p, b1p, W2p, b2p):
    del Wg0, bg0

    ei_r = edge_index.reshape(2, _NW, _NSUP, _SCH, _CH)
    ew_r = edge_weight.reshape(_NW, _NSUP, _EPS)

    ones80 = jnp.ones((_CH,), _F32)
    zeros1d = jnp.zeros((_DEGPAD,), _F32)
    zeros2d = jnp.zeros((_RPT, _OUT), _F32)
    gidx = jnp.full((_CH,), _PADROW, jnp.int32)

    deg = _degrees(ei_r, ones80, zeros1d, gidx)
    deg4 = deg.reshape(_NC, 2, _DEGPAD, 1)

    zs = _dense(h, W1r, b1r.reshape(1, _HID), W2r, b2r.reshape(1, _OUT),
                Wg1, deg4)
    aggs = _edges(zs, ei_r, ew_r, zeros2d, gidx)

    hf, po = _final(aggs, deg4, bg1.reshape(1, _OUT), protein_h,
                    W1p, b1p.reshape(1, _HID), W2p, b2p.reshape(1, _OUT))
    return (hf, po)


# speedup vs baseline: 20.1422x; 1.0962x over previous
import functools

import jax
import jax.numpy as jnp
from jax import lax
from jax.experimental import pallas as pl
from jax.experimental.pallas import tpu as pltpu
from jax.experimental.pallas import tpu_sc as plsc

_N = 10000
_E = 320000
_DIN = 128
_HID = 256
_OUT = 128
_PIN = 1280
_NPROT = 19

_NC = 2
_NS = 16
_NW = _NC * _NS
_EPW = _E // _NW
_CH = 40
_NCHUNK = _EPW // _CH
_NPAD = 10240
_RPT = _NPAD // _NS
_ZCH = 128
_DEGPAD = 10240
_DEGTOT = 2 * _DEGPAD
_DPT = _DEGPAD // _NS
_SCH = 50
_NSUP = _NCHUNK // _SCH
_EPS = _SCH * _CH
_K = 5
_GPS = _SCH // _K
_PADROW = 10016

_F32 = jnp.float32
_HIGH = lax.Precision.HIGHEST

def _mesh():
    return plsc.VectorSubcoreMesh(core_axis_name="c", subcore_axis_name="s",
                                  num_cores=_NC, num_subcores=_NS)


def _degree_body(ei_hbm, ones_hbm, zeros_hbm, gidx_hbm, out_hbm, idx_v,
                 ones_v, buf_v, gidx_v, dego_sh, degi_sh, sem0, sem1, sem2,
                 sem3):
    cid = lax.axis_index("c")
    sid = lax.axis_index("s")
    wid = sid * _NC + cid

    pltpu.sync_copy(ei_hbm.at[0, wid], idx_v.at[0])
    pltpu.sync_copy(ei_hbm.at[1, wid], idx_v.at[1])
    pltpu.sync_copy(ones_hbm, ones_v)
    pltpu.sync_copy(gidx_hbm, gidx_v)

    pltpu.sync_copy(zeros_hbm.at[pl.ds(sid * _DPT, _DPT)], buf_v)
    pltpu.sync_copy(buf_v, dego_sh.at[pl.ds(sid * _DPT, _DPT)])
    pltpu.sync_copy(buf_v, degi_sh.at[pl.ds(sid * _DPT, _DPT)])
    plsc.subcore_barrier()

    pltpu.async_copy(ones_v, dego_sh.at[gidx_v], sem0, add=True)
    pltpu.async_copy(ones_v, degi_sh.at[gidx_v], sem1, add=True)
    pltpu.async_copy(ones_v, dego_sh.at[gidx_v], sem2, add=True)
    pltpu.async_copy(ones_v, degi_sh.at[gidx_v], sem3, add=True)

    def _scat_s(s, _):
        def _scat(jj, _):
            j0 = 2 * jj
            j1 = j0 + 1
            pltpu.make_async_copy(ones_v, dego_sh.at[gidx_v], sem0).wait()
            pltpu.async_copy(ones_v, dego_sh.at[idx_v.at[0, s, j0]], sem0,
                             add=True)
            pltpu.make_async_copy(ones_v, degi_sh.at[gidx_v], sem1).wait()
            pltpu.async_copy(ones_v, degi_sh.at[idx_v.at[1, s, j0]], sem1,
                             add=True)
            pltpu.make_async_copy(ones_v, dego_sh.at[gidx_v], sem2).wait()
            pltpu.async_copy(ones_v, dego_sh.at[idx_v.at[0, s, j1]], sem2,
                             add=True)
            pltpu.make_async_copy(ones_v, degi_sh.at[gidx_v], sem3).wait()
            pltpu.async_copy(ones_v, degi_sh.at[idx_v.at[1, s, j1]], sem3,
                             add=True)
            return ()
        lax.fori_loop(0, _SCH // 2, _scat, ())
        return ()
    lax.fori_loop(0, _NSUP, _scat_s, ())
    pltpu.make_async_copy(ones_v, dego_sh.at[gidx_v], sem0).wait()
    pltpu.make_async_copy(ones_v, degi_sh.at[gidx_v], sem1).wait()
    pltpu.make_async_copy(ones_v, dego_sh.at[gidx_v], sem2).wait()
    pltpu.make_async_copy(ones_v, degi_sh.at[gidx_v], sem3).wait()
    plsc.subcore_barrier()

    pltpu.sync_copy(dego_sh.at[pl.ds(sid * _DPT, _DPT)], buf_v)
    pltpu.sync_copy(buf_v, out_hbm.at[cid, 0, pl.ds(sid * _DPT, _DPT)])
    pltpu.sync_copy(degi_sh.at[pl.ds(sid * _DPT, _DPT)], buf_v)
    pltpu.sync_copy(buf_v, out_hbm.at[cid, 1, pl.ds(sid * _DPT, _DPT)])


def _degrees(ei_r, ones80, zeros1d, gidx):
    return pl.kernel(
        _degree_body,
        out_type=jax.ShapeDtypeStruct((_NC, 2, _DEGPAD), _F32),
        mesh=_mesh(),
        compiler_params=pltpu.CompilerParams(needs_layout_passes=False),
        scratch_types=[
            pltpu.VMEM((2, _NSUP, _SCH, _CH), jnp.int32),
            pltpu.VMEM((_CH,), _F32),
            pltpu.VMEM((_DPT,), _F32),
            pltpu.VMEM((_CH,), jnp.int32),
            pltpu.VMEM_SHARED((_DEGPAD,), _F32),
            pltpu.VMEM_SHARED((_DEGPAD,), _F32),
            pltpu.SemaphoreType.DMA,
            pltpu.SemaphoreType.DMA,
            pltpu.SemaphoreType.DMA,
            pltpu.SemaphoreType.DMA,
        ],
    )(ei_r, ones80, zeros1d, gidx)


_BD_ROWS = 1000
_BD_GRID = _N // _BD_ROWS


def _dense_body(h_ref, w1_ref, b1_ref, w2_ref, b2_ref, wg_ref,
                z_ref, wf_ref, bf_ref):
    i = pl.program_id(0)

    @pl.when(i == 0)
    def _():
        wf_ref[...] = jnp.dot(w2_ref[...], wg_ref[...],
                              preferred_element_type=_F32, precision=_HIGH)
        bf_ref[...] = jnp.dot(b2_ref[...], wg_ref[...],
                              preferred_element_type=_F32, precision=_HIGH)

    hm = jnp.dot(h_ref[...], w1_ref[...],
                 preferred_element_type=_F32) + b1_ref[...]
    hm = jnp.maximum(hm, 0.0)
    z_ref[...] = jnp.dot(hm, wf_ref[...],
                         preferred_element_type=_F32) + bf_ref[...]


def _scale_body(z_ref, deg_ref, zs_ref):
    d = deg_ref[0, 0] + deg_ref[1, 0]
    zs_ref[...] = z_ref[...] * lax.rsqrt(jnp.maximum(d, 1.0))


def _scale_z(z, deg4):
    return pl.pallas_call(
        _scale_body,
        grid=(_BD_GRID,),
        in_specs=[
            pl.BlockSpec((_BD_ROWS, _OUT), lambda i: (i, 0)),
            pl.BlockSpec((_NC, 1, _BD_ROWS, 1), lambda i: (0, 0, i, 0)),
        ],
        out_specs=pl.BlockSpec((_BD_ROWS, _OUT), lambda i: (i, 0)),
        out_shape=jax.ShapeDtypeStruct((_N, _OUT), _F32),
    )(z, deg4[:, :1])


def _dense(h, W1r, b1r, W2r, b2r, Wg1):
    return pl.pallas_call(
        _dense_body,
        grid=(_BD_GRID,),
        in_specs=[
            pl.BlockSpec((_BD_ROWS, _DIN), lambda i: (i, 0)),
            pl.BlockSpec((_DIN, _HID), lambda i: (0, 0)),
            pl.BlockSpec((1, _HID), lambda i: (0, 0)),
            pl.BlockSpec((_HID, _OUT), lambda i: (0, 0)),
            pl.BlockSpec((1, _OUT), lambda i: (0, 0)),
            pl.BlockSpec((_OUT, _OUT), lambda i: (0, 0)),
        ],
        out_specs=pl.BlockSpec((_BD_ROWS, _OUT), lambda i: (i, 0)),
        out_shape=jax.ShapeDtypeStruct((_N, _OUT), _F32),
        scratch_shapes=[
            pltpu.VMEM((_HID, _OUT), _F32),
            pltpu.VMEM((1, _OUT), _F32),
        ],
    )(h, W1r, b1r, W2r, b2r, Wg1)


def _edge_kernel_body(zs_hbm, ei_hbm, ew_hbm, zeros_hbm, gidx_hbm, out_hbm,
                      src_i, dst_i, ew_v, rows, gidx_v,
                      agg_sh, gs0, gs1, gs2, gs3, gs4, ss0, ss1, ss2, ss3,
                      ss4):
    cid = lax.axis_index("c")
    sid = lax.axis_index("s")
    wid = sid * _NC + cid
    gs = (gs0, gs1, gs2, gs3, gs4)
    ss = (ss0, ss1, ss2, ss3, ss4)

    pltpu.sync_copy(gidx_hbm, gidx_v)

    pltpu.sync_copy(zeros_hbm, agg_sh.at[pl.ds(sid * _RPT, _RPT)])

    for k in range(_K):
        pltpu.async_copy(rows.at[k], agg_sh.at[gidx_v], ss[k], add=True)
    plsc.subcore_barrier()

    def _sup(s, _):
        pltpu.sync_copy(ei_hbm.at[0, wid, s], src_i)
        pltpu.sync_copy(ei_hbm.at[1, wid, s], dst_i)
        pltpu.sync_copy(ew_hbm.at[wid, s], ew_v)

        for k in range(3):
            pltpu.make_async_copy(rows.at[k], agg_sh.at[gidx_v],
                                  ss[k]).wait()
            pltpu.async_copy(zs_hbm.at[src_i.at[k]], rows.at[k], gs[k])

        def _prefetch(g, kp):
            jn = (g + 1) * _K + kp

            @pl.when(g + 1 < _GPS)
            def _():
                pltpu.make_async_copy(rows.at[kp], agg_sh.at[gidx_v],
                                      ss[kp]).wait()
                pltpu.async_copy(zs_hbm.at[src_i.at[jn]], rows.at[kp],
                                 gs[kp])

        def _grp(g, _):
            for k in range(_K):
                j = g * _K + k
                if k < 2:
                    b = k + 3
                    jb = g * _K + b
                    pltpu.make_async_copy(rows.at[b], agg_sh.at[gidx_v],
                                          ss[b]).wait()
                    pltpu.async_copy(zs_hbm.at[src_i.at[jb]], rows.at[b],
                                     gs[b])
                pltpu.make_async_copy(zs_hbm.at[src_i.at[j]], rows.at[k],
                                      gs[k]).wait()

                def _scale(e4, _, k=k, j=j):
                    for u in range(4):
                        e = e4 * 4 + u
                        ewv = plsc.load_gather(
                            ew_v, [jnp.full((16,), j * _CH + e, jnp.int32)])
                        for gg in range(8):
                            sl = pl.ds(gg * 16, 16)
                            rows[k, e, sl] = rows[k, e, sl] * ewv
                    return ()
                lax.fori_loop(0, _CH // 4, _scale, ())

                pltpu.async_copy(rows.at[k], agg_sh.at[dst_i.at[j]], ss[k],
                                 add=True)
                if k >= 2:
                    _prefetch(g, k - 2)
            return ()
        lax.fori_loop(0, _GPS, _grp, ())
        return ()
    lax.fori_loop(0, _NSUP, _sup, ())

    for k in range(_K):
        pltpu.make_async_copy(rows.at[k], agg_sh.at[gidx_v], ss[k]).wait()
    plsc.subcore_barrier()

    pltpu.sync_copy(agg_sh.at[pl.ds(sid * _RPT, _RPT)],
                    out_hbm.at[cid, pl.ds(sid * _RPT, _RPT)])


def _edges(zs, ei_r, ew_r, zeros2d, gidx):
    return pl.kernel(
        _edge_kernel_body,
        out_type=jax.ShapeDtypeStruct((_NC, _NPAD, _OUT), _F32),
        mesh=_mesh(),
        compiler_params=pltpu.CompilerParams(needs_layout_passes=False),
        scratch_types=[
            pltpu.VMEM((_SCH, _CH), jnp.int32),
            pltpu.VMEM((_SCH, _CH), jnp.int32),
            pltpu.VMEM((_EPS,), _F32),
            pltpu.VMEM((_K, _CH, _OUT), _F32),
            pltpu.VMEM((_CH,), jnp.int32),
            pltpu.VMEM_SHARED((_NPAD, _OUT), _F32),
        ] + [pltpu.SemaphoreType.DMA] * (2 * _K),
    )(zs, ei_r, ew_r, zeros2d, gidx)


def _final_body(a0_ref, a1_ref, deg_ref, bg_ref, ph_ref, w1p_ref, b1p_ref,
                w2p_ref, b2p_ref, hf_ref, po_ref):
    i = pl.program_id(0)

    a = a0_ref[0] + a1_ref[0]
    d = deg_ref[0, 0] + deg_ref[1, 0]
    hl = a * lax.rsqrt(jnp.maximum(d, 1.0)) + bg_ref[...]
    h1 = jnp.where(hl >= 0.0, hl, 0.01 * hl)
    part = jnp.sum(h1, axis=0, keepdims=True)

    @pl.when(i == 0)
    def _():
        hf_ref[...] = jnp.zeros_like(hf_ref)
        hm = jnp.dot(ph_ref[...], w1p_ref[...], preferred_element_type=_F32,
                     precision=_HIGH) + b1p_ref[...]
        hm = jnp.maximum(hm, 0.0)
        po_ref[...] = jnp.dot(hm, w2p_ref[...], preferred_element_type=_F32,
                              precision=_HIGH) + b2p_ref[...]

    hf_ref[...] += part

    @pl.when(i == _BD_GRID - 1)
    def _():
        hf_ref[...] = hf_ref[...] * (1.0 / _N)


def _final(aggs, deg4, bg1, ph, W1p, b1p, W2p, b2p):
    return pl.pallas_call(
        _final_body,
        grid=(_BD_GRID,),
        in_specs=[
            pl.BlockSpec((1, _BD_ROWS, _OUT), lambda i: (0, i, 0)),
            pl.BlockSpec((1, _BD_ROWS, _OUT), lambda i: (1, i, 0)),
            pl.BlockSpec((_NC, 1, _BD_ROWS, 1), lambda i: (0, 0, i, 0)),
            pl.BlockSpec((1, _OUT), lambda i: (0, 0)),
            pl.BlockSpec((_NPROT, _PIN), lambda i: (0, 0)),
            pl.BlockSpec((_PIN, _HID), lambda i: (0, 0)),
            pl.BlockSpec((1, _HID), lambda i: (0, 0)),
            pl.BlockSpec((_HID, _OUT), lambda i: (0, 0)),
            pl.BlockSpec((1, _OUT), lambda i: (0, 0)),
        ],
        out_specs=[
            pl.BlockSpec((1, _OUT), lambda i: (0, 0)),
            pl.BlockSpec((_NPROT, _OUT), lambda i: (0, 0)),
        ],
        out_shape=[
            jax.ShapeDtypeStruct((1, _OUT), _F32),
            jax.ShapeDtypeStruct((_NPROT, _OUT), _F32),
        ],
    )(aggs, aggs, deg4[:, 1:], bg1, ph, W1p, b1p, W2p, b2p)


def kernel(h, edge_index, edge_weight, protein_h, W1r, b1r, W2r, b2r,
           Wg0, bg0, Wg1, bg1, W1p, b1p, W2p, b2p):
    del Wg0, bg0

    ei_r = edge_index.reshape(2, _NW, _NSUP, _SCH, _CH)
    ew_r = edge_weight.reshape(_NW, _NSUP, _EPS)

    ones80 = jnp.ones((_CH,), _F32)
    zeros1d = jnp.zeros((_DEGPAD,), _F32)
    zeros2d = jnp.zeros((_RPT, _OUT), _F32)
    gidx = jnp.full((_CH,), _PADROW, jnp.int32)

    deg = _degrees(ei_r, ones80, zeros1d, gidx)
    deg4 = deg.reshape(_NC, 2, _DEGPAD, 1)

    z = _dense(h, W1r, b1r.reshape(1, _HID), W2r, b2r.reshape(1, _OUT),
               Wg1)
    zs = _scale_z(z, deg4)
    aggs = _edges(zs, ei_r, ew_r, zeros2d, gidx)

    hf, po = _final(aggs, deg4, bg1.reshape(1, _OUT), protein_h,
                    W1p, b1p.reshape(1, _HID), W2p, b2p.reshape(1, _OUT))
    return (hf, po)


# speedup vs baseline: 20.1567x; 1.0007x over previous
import jax
import jax.numpy as jnp
from jax import lax
from jax.experimental import pallas as pl
from jax.experimental.pallas import tpu as pltpu
from jax.experimental.pallas import tpu_sc as plsc

_N = 10000
_E = 320000
_DIN = 128
_HID = 256
_OUT = 128
_PIN = 1280
_NPROT = 19

_NC = 2
_NS = 16
_NW = _NC * _NS
_EPW = _E // _NW
_CH = 40
_NCHUNK = _EPW // _CH
_NPAD = 10240
_RPT = _NPAD // _NS
_DEGPAD = 10240
_DPT = _DEGPAD // _NS
_SCH = 50
_NSUP = _NCHUNK // _SCH
_EPS = _SCH * _CH
_K = 5
_GPS = _SCH // _K
_PADROW = 10016

_F32 = jnp.float32
_HIGH = lax.Precision.HIGHEST

def _mesh():
    return plsc.VectorSubcoreMesh(core_axis_name="c", subcore_axis_name="s",
                                  num_cores=_NC, num_subcores=_NS)


def _degree_body(ei_hbm, ones_hbm, zeros_hbm, gidx_hbm, out_hbm, idx_v,
                 ones_v, buf_v, gidx_v, dego_sh, degi_sh, sem0, sem1, sem2,
                 sem3):
    cid = lax.axis_index("c")
    sid = lax.axis_index("s")
    wid = sid * _NC + cid

    pltpu.sync_copy(ei_hbm.at[0, wid], idx_v.at[0])
    pltpu.sync_copy(ei_hbm.at[1, wid], idx_v.at[1])
    pltpu.sync_copy(ones_hbm, ones_v)
    pltpu.sync_copy(gidx_hbm, gidx_v)

    pltpu.sync_copy(zeros_hbm.at[pl.ds(sid * _DPT, _DPT)], buf_v)
    pltpu.sync_copy(buf_v, dego_sh.at[pl.ds(sid * _DPT, _DPT)])
    pltpu.sync_copy(buf_v, degi_sh.at[pl.ds(sid * _DPT, _DPT)])
    plsc.subcore_barrier()

    pltpu.async_copy(ones_v, dego_sh.at[gidx_v], sem0, add=True)
    pltpu.async_copy(ones_v, degi_sh.at[gidx_v], sem1, add=True)
    pltpu.async_copy(ones_v, dego_sh.at[gidx_v], sem2, add=True)
    pltpu.async_copy(ones_v, degi_sh.at[gidx_v], sem3, add=True)

    def _scat_s(s, _):
        def _scat(jj, _):
            j0 = 2 * jj
            j1 = j0 + 1
            pltpu.make_async_copy(ones_v, dego_sh.at[gidx_v], sem0).wait()
            pltpu.async_copy(ones_v, dego_sh.at[idx_v.at[0, s, j0]], sem0,
                             add=True)
            pltpu.make_async_copy(ones_v, degi_sh.at[gidx_v], sem1).wait()
            pltpu.async_copy(ones_v, degi_sh.at[idx_v.at[1, s, j0]], sem1,
                             add=True)
            pltpu.make_async_copy(ones_v, dego_sh.at[gidx_v], sem2).wait()
            pltpu.async_copy(ones_v, dego_sh.at[idx_v.at[0, s, j1]], sem2,
                             add=True)
            pltpu.make_async_copy(ones_v, degi_sh.at[gidx_v], sem3).wait()
            pltpu.async_copy(ones_v, degi_sh.at[idx_v.at[1, s, j1]], sem3,
                             add=True)
            return ()
        lax.fori_loop(0, _SCH // 2, _scat, ())
        return ()
    lax.fori_loop(0, _NSUP, _scat_s, ())
    pltpu.make_async_copy(ones_v, dego_sh.at[gidx_v], sem0).wait()
    pltpu.make_async_copy(ones_v, degi_sh.at[gidx_v], sem1).wait()
    pltpu.make_async_copy(ones_v, dego_sh.at[gidx_v], sem2).wait()
    pltpu.make_async_copy(ones_v, degi_sh.at[gidx_v], sem3).wait()
    plsc.subcore_barrier()

    pltpu.sync_copy(dego_sh.at[pl.ds(sid * _DPT, _DPT)], buf_v)
    pltpu.sync_copy(buf_v, out_hbm.at[cid, 0, pl.ds(sid * _DPT, _DPT)])
    pltpu.sync_copy(degi_sh.at[pl.ds(sid * _DPT, _DPT)], buf_v)
    pltpu.sync_copy(buf_v, out_hbm.at[cid, 1, pl.ds(sid * _DPT, _DPT)])


def _degrees(ei_r, ones80, zeros1d, gidx):
    return pl.kernel(
        _degree_body,
        out_type=jax.ShapeDtypeStruct((_NC, 2, _DEGPAD), _F32),
        mesh=_mesh(),
        compiler_params=pltpu.CompilerParams(needs_layout_passes=False),
        scratch_types=[
            pltpu.VMEM((2, _NSUP, _SCH, _CH), jnp.int32),
            pltpu.VMEM((_CH,), _F32),
            pltpu.VMEM((_DPT,), _F32),
            pltpu.VMEM((_CH,), jnp.int32),
            pltpu.VMEM_SHARED((_DEGPAD,), _F32),
            pltpu.VMEM_SHARED((_DEGPAD,), _F32),
            pltpu.SemaphoreType.DMA,
            pltpu.SemaphoreType.DMA,
            pltpu.SemaphoreType.DMA,
            pltpu.SemaphoreType.DMA,
        ],
    )(ei_r, ones80, zeros1d, gidx)


_BD_ROWS = 1000
_BD_GRID = _N // _BD_ROWS


def _dense_body(h_ref, w1_ref, b1_ref, w2_ref, b2_ref, wg_ref,
                z_ref, wf_ref, bf_ref):
    i = pl.program_id(0)

    @pl.when(i == 0)
    def _():
        wf_ref[...] = jnp.dot(w2_ref[...], wg_ref[...],
                              preferred_element_type=_F32, precision=_HIGH)
        bf_ref[...] = jnp.dot(b2_ref[...], wg_ref[...],
                              preferred_element_type=_F32, precision=_HIGH)

    hm = jnp.dot(h_ref[...], w1_ref[...],
                 preferred_element_type=_F32) + b1_ref[...]
    hm = jnp.maximum(hm, 0.0)
    z_ref[...] = jnp.dot(hm, wf_ref[...],
                         preferred_element_type=_F32) + bf_ref[...]


def _scale_body(z_ref, deg_ref, zs_ref):
    d = deg_ref[0, 0] + deg_ref[1, 0]
    zs_ref[...] = z_ref[...] * lax.rsqrt(jnp.maximum(d, 1.0))


def _scale_z(z, deg4):
    return pl.pallas_call(
        _scale_body,
        grid=(_BD_GRID,),
        in_specs=[
            pl.BlockSpec((_BD_ROWS, _OUT), lambda i: (i, 0)),
            pl.BlockSpec((_NC, 1, _BD_ROWS, 1), lambda i: (0, 0, i, 0)),
        ],
        out_specs=pl.BlockSpec((_BD_ROWS, _OUT), lambda i: (i, 0)),
        out_shape=jax.ShapeDtypeStruct((_N, _OUT), _F32),
    )(z, deg4[:, :1])


def _dense(h, W1r, b1r, W2r, b2r, Wg1):
    return pl.pallas_call(
        _dense_body,
        grid=(_BD_GRID,),
        in_specs=[
            pl.BlockSpec((_BD_ROWS, _DIN), lambda i: (i, 0)),
            pl.BlockSpec((_DIN, _HID), lambda i: (0, 0)),
            pl.BlockSpec((1, _HID), lambda i: (0, 0)),
            pl.BlockSpec((_HID, _OUT), lambda i: (0, 0)),
            pl.BlockSpec((1, _OUT), lambda i: (0, 0)),
            pl.BlockSpec((_OUT, _OUT), lambda i: (0, 0)),
        ],
        out_specs=pl.BlockSpec((_BD_ROWS, _OUT), lambda i: (i, 0)),
        out_shape=jax.ShapeDtypeStruct((_N, _OUT), _F32),
        scratch_shapes=[
            pltpu.VMEM((_HID, _OUT), _F32),
            pltpu.VMEM((1, _OUT), _F32),
        ],
    )(h, W1r, b1r, W2r, b2r, Wg1)


def _edge_kernel_body(zs_hbm, ei_hbm, ew_hbm, zeros_hbm, gidx_hbm, out_hbm,
                      src_i, dst_i, ew_v, rows, gidx_v,
                      agg_sh, gs0, gs1, gs2, gs3, gs4, ss0, ss1, ss2, ss3,
                      ss4):
    cid = lax.axis_index("c")
    sid = lax.axis_index("s")
    wid = sid * _NC + cid
    gs = (gs0, gs1, gs2, gs3, gs4)
    ss = (ss0, ss1, ss2, ss3, ss4)

    pltpu.sync_copy(gidx_hbm, gidx_v)

    pltpu.sync_copy(zeros_hbm, agg_sh.at[pl.ds(sid * _RPT, _RPT)])

    for k in range(_K):
        pltpu.async_copy(rows.at[k], agg_sh.at[gidx_v], ss[k], add=True)
    plsc.subcore_barrier()

    def _sup(s, _):
        pltpu.sync_copy(ei_hbm.at[0, wid, s], src_i)
        pltpu.sync_copy(ei_hbm.at[1, wid, s], dst_i)
        pltpu.sync_copy(ew_hbm.at[wid, s], ew_v)

        for k in range(3):
            pltpu.make_async_copy(rows.at[k], agg_sh.at[gidx_v],
                                  ss[k]).wait()
            pltpu.async_copy(zs_hbm.at[src_i.at[k]], rows.at[k], gs[k])

        def _prefetch(g, kp):
            jn = (g + 1) * _K + kp

            @pl.when(g + 1 < _GPS)
            def _():
                pltpu.make_async_copy(rows.at[kp], agg_sh.at[gidx_v],
                                      ss[kp]).wait()
                pltpu.async_copy(zs_hbm.at[src_i.at[jn]], rows.at[kp],
                                 gs[kp])

        def _grp(g, _):
            for k in range(_K):
                j = g * _K + k
                if k < 2:
                    b = k + 3
                    jb = g * _K + b
                    pltpu.make_async_copy(rows.at[b], agg_sh.at[gidx_v],
                                          ss[b]).wait()
                    pltpu.async_copy(zs_hbm.at[src_i.at[jb]], rows.at[b],
                                     gs[b])
                pltpu.make_async_copy(zs_hbm.at[src_i.at[j]], rows.at[k],
                                      gs[k]).wait()

                def _scale(e4, _, k=k, j=j):
                    for u in range(4):
                        e = e4 * 4 + u
                        ewv = plsc.load_gather(
                            ew_v, [jnp.full((16,), j * _CH + e, jnp.int32)])
                        for gg in range(8):
                            sl = pl.ds(gg * 16, 16)
                            rows[k, e, sl] = rows[k, e, sl] * ewv
                    return ()
                lax.fori_loop(0, _CH // 4, _scale, ())

                pltpu.async_copy(rows.at[k], agg_sh.at[dst_i.at[j]], ss[k],
                                 add=True)
                if k >= 2:
                    _prefetch(g, k - 2)
            return ()
        lax.fori_loop(0, _GPS, _grp, ())
        return ()
    lax.fori_loop(0, _NSUP, _sup, ())

    for k in range(_K):
        pltpu.make_async_copy(rows.at[k], agg_sh.at[gidx_v], ss[k]).wait()
    plsc.subcore_barrier()

    pltpu.sync_copy(agg_sh.at[pl.ds(sid * _RPT, _RPT)],
                    out_hbm.at[cid, pl.ds(sid * _RPT, _RPT)])


def _edges(zs, ei_r, ew_r, zeros2d, gidx):
    return pl.kernel(
        _edge_kernel_body,
        out_type=jax.ShapeDtypeStruct((_NC, _NPAD, _OUT), _F32),
        mesh=_mesh(),
        compiler_params=pltpu.CompilerParams(needs_layout_passes=False),
        scratch_types=[
            pltpu.VMEM((_SCH, _CH), jnp.int32),
            pltpu.VMEM((_SCH, _CH), jnp.int32),
            pltpu.VMEM((_EPS,), _F32),
            pltpu.VMEM((_K, _CH, _OUT), _F32),
            pltpu.VMEM((_CH,), jnp.int32),
            pltpu.VMEM_SHARED((_NPAD, _OUT), _F32),
        ] + [pltpu.SemaphoreType.DMA] * (2 * _K),
    )(zs, ei_r, ew_r, zeros2d, gidx)


def _final_body(a0_ref, a1_ref, deg_ref, bg_ref, ph_ref, w1p_ref, b1p_ref,
                w2p_ref, b2p_ref, hf_ref, po_ref):
    i = pl.program_id(0)

    a = a0_ref[0] + a1_ref[0]
    d = deg_ref[0, 0] + deg_ref[1, 0]
    hl = a * lax.rsqrt(jnp.maximum(d, 1.0)) + bg_ref[...]
    h1 = jnp.where(hl >= 0.0, hl, 0.01 * hl)
    part = jnp.sum(h1, axis=0, keepdims=True)

    @pl.when(i == 0)
    def _():
        hf_ref[...] = jnp.zeros_like(hf_ref)
        hm = jnp.dot(ph_ref[...], w1p_ref[...], preferred_element_type=_F32,
                     precision=_HIGH) + b1p_ref[...]
        hm = jnp.maximum(hm, 0.0)
        po_ref[...] = jnp.dot(hm, w2p_ref[...], preferred_element_type=_F32,
                              precision=_HIGH) + b2p_ref[...]

    hf_ref[...] += part

    @pl.when(i == _BD_GRID - 1)
    def _():
        hf_ref[...] = hf_ref[...] * (1.0 / _N)


def _final(aggs, deg4, bg1, ph, W1p, b1p, W2p, b2p):
    return pl.pallas_call(
        _final_body,
        grid=(_BD_GRID,),
        in_specs=[
            pl.BlockSpec((1, _BD_ROWS, _OUT), lambda i: (0, i, 0)),
            pl.BlockSpec((1, _BD_ROWS, _OUT), lambda i: (1, i, 0)),
            pl.BlockSpec((_NC, 1, _BD_ROWS, 1), lambda i: (0, 0, i, 0)),
            pl.BlockSpec((1, _OUT), lambda i: (0, 0)),
            pl.BlockSpec((_NPROT, _PIN), lambda i: (0, 0)),
            pl.BlockSpec((_PIN, _HID), lambda i: (0, 0)),
            pl.BlockSpec((1, _HID), lambda i: (0, 0)),
            pl.BlockSpec((_HID, _OUT), lambda i: (0, 0)),
            pl.BlockSpec((1, _OUT), lambda i: (0, 0)),
        ],
        out_specs=[
            pl.BlockSpec((1, _OUT), lambda i: (0, 0)),
            pl.BlockSpec((_NPROT, _OUT), lambda i: (0, 0)),
        ],
        out_shape=[
            jax.ShapeDtypeStruct((1, _OUT), _F32),
            jax.ShapeDtypeStruct((_NPROT, _OUT), _F32),
        ],
    )(aggs, aggs, deg4[:, 1:], bg1, ph, W1p, b1p, W2p, b2p)


def kernel(h, edge_index, edge_weight, protein_h, W1r, b1r, W2r, b2r,
           Wg0, bg0, Wg1, bg1, W1p, b1p, W2p, b2p):
    del Wg0, bg0

    ei_r = edge_index.reshape(2, _NW, _NSUP, _SCH, _CH)
    ew_r = edge_weight.reshape(_NW, _NSUP, _EPS)

    ones80 = jnp.ones((_CH,), _F32)
    zeros1d = jnp.zeros((_DEGPAD,), _F32)
    zeros2d = jnp.zeros((_RPT, _OUT), _F32)
    gidx = jnp.full((_CH,), _PADROW, jnp.int32)

    deg = _degrees(ei_r, ones80, zeros1d, gidx)
    deg4 = deg.reshape(_NC, 2, _DEGPAD, 1)

    z = _dense(h, W1r, b1r.reshape(1, _HID), W2r, b2r.reshape(1, _OUT),
               Wg1)
    zs = _scale_z(z, deg4)
    aggs = _edges(zs, ei_r, ew_r, zeros2d, gidx)

    hf, po = _final(aggs, deg4, bg1.reshape(1, _OUT), protein_h,
                    W1p, b1p.reshape(1, _HID), W2p, b2p.reshape(1, _OUT))
    return (hf, po)
